# baseline (speedup 1.0000x reference)
import functools

import jax
import jax.numpy as jnp
from jax import lax
from jax.experimental import pallas as pl
from jax.experimental.pallas import tpu as pltpu
from jax.experimental.pallas import tpu_sc as plsc

N = 10000
R = 4
E = 80000
D = 128

NCORE = 2
NSUB = 16
LANES = 16
REL_PER_CORE = R // NCORE
NP = 10240
ROWS_PER_TILE = NP // NSUB
CH = 64
CPT = 80
EPAD = NSUB * CPT * CH
PAD_DST = N
ZROWS = 128
DST_SHIFT = 14
SRC_MASK = (1 << DST_SHIFT) - 1


def _zero_buf_2d(buf, nrows, ncols):
    zv = jnp.zeros((LANES,), jnp.float32)

    def row(i, _):
        def col(j, _):
            buf[i, pl.ds(j * LANES, LANES)] = zv
            return 0
        return lax.fori_loop(0, ncols // LANES, col, 0)

    lax.fori_loop(0, nrows, row, 0)


def _unpack_chunk(comb_all, j, idx_s, idx_d):
    base = j * CH
    for t in range(CH // LANES):
        v = comb_all[pl.ds(base + t * LANES, LANES)]
        if idx_s is not None:
            idx_s[pl.ds(t * LANES, LANES)] = v & SRC_MASK
        idx_d[pl.ds(t * LANES, LANES)] = lax.shift_right_logical(v, DST_SHIFT)


def _sc_agg_body(feat, combm, agg_out,
                 comb_all, idx_s0, idx_s1, idx_d0, idx_d1,
                 rows0, rows1, z128, sem0, sem1, acc_sh):
    cid = lax.axis_index("c")
    sid = lax.axis_index("s")

    _zero_buf_2d(z128, ZROWS, D)
    row0 = sid * ROWS_PER_TILE
    idx_s = (idx_s0, idx_s1)
    idx_d = (idx_d0, idx_d1)
    rows = (rows0, rows1)
    sems = (sem0, sem1)

    for k in range(REL_PER_CORE):
        r = cid * REL_PER_CORE + k

        for m in range(ROWS_PER_TILE // ZROWS):
            pltpu.sync_copy(z128, acc_sh.at[pl.ds(row0 + m * ZROWS, ZROWS)])
        plsc.subcore_barrier()

        ebase = r * EPAD + sid * CPT * CH
        pltpu.sync_copy(combm.at[pl.ds(ebase, CPT * CH)], comb_all)

        for b in range(2):
            _unpack_chunk(comb_all, b, idx_s[b], idx_d[b])
            pltpu.async_copy(feat.at[idx_s[b]], rows[b], sems[b])

        def pair(i, _):
            for b in range(2):
                j = 2 * i + b
                pltpu.make_async_copy(
                    feat.at[idx_s[b]], rows[b], sems[b]).wait()
                pltpu.sync_copy(rows[b], acc_sh.at[idx_d[b]], add=True)

                @pl.when(j < CPT - 2)
                def _():
                    _unpack_chunk(comb_all, j + 2, idx_s[b], idx_d[b])
                    pltpu.async_copy(feat.at[idx_s[b]], rows[b], sems[b])
            return 0

        lax.fori_loop(0, CPT // 2, pair, 0)
        plsc.subcore_barrier()

        pltpu.sync_copy(acc_sh.at[pl.ds(row0, ROWS_PER_TILE)],
                        agg_out.at[pl.ds(r * NP + row0, ROWS_PER_TILE)])


def _sc_deg_body(combm, deg_out, comb_all, idx_d0, ones_b, z128, dacc_sh):
    cid = lax.axis_index("c")
    sid = lax.axis_index("s")

    _zero_buf_2d(z128, ZROWS, D)
    ov = jnp.ones((LANES,), jnp.float32)

    def ones_row(i, _):
        def ones_col(j, _):
            ones_b[i, pl.ds(j * LANES, LANES)] = ov
            return 0
        return lax.fori_loop(0, D // LANES, ones_col, 0)

    lax.fori_loop(0, CH, ones_row, 0)

    row0 = sid * ROWS_PER_TILE

    for k in range(REL_PER_CORE):
        r = cid * REL_PER_CORE + k

        for m in range(ROWS_PER_TILE // ZROWS):
            pltpu.sync_copy(z128, dacc_sh.at[pl.ds(row0 + m * ZROWS, ZROWS)])
        plsc.subcore_barrier()

        ebase = r * EPAD + sid * CPT * CH
        pltpu.sync_copy(combm.at[pl.ds(ebase, CPT * CH)], comb_all)

        def chunk(j, _):
            _unpack_chunk(comb_all, j, None, idx_d0)
            pltpu.sync_copy(ones_b, dacc_sh.at[idx_d0], add=True)
            return 0

        lax.fori_loop(0, CPT, chunk, 0)
        plsc.subcore_barrier()

        pltpu.sync_copy(dacc_sh.at[pl.ds(row0, ROWS_PER_TILE)],
                        deg_out.at[pl.ds(r * NP + row0, ROWS_PER_TILE)])


@functools.lru_cache(maxsize=None)
def _make_sc_agg():
    return pl.kernel(
        _sc_agg_body,
        out_type=[jax.ShapeDtypeStruct((R * NP, D), jnp.float32)],
        mesh=plsc.VectorSubcoreMesh(core_axis_name="c", subcore_axis_name="s"),
        scratch_types=[
            pltpu.VMEM((CPT * CH,), jnp.int32),
            pltpu.VMEM((CH,), jnp.int32),
            pltpu.VMEM((CH,), jnp.int32),
            pltpu.VMEM((CH,), jnp.int32),
            pltpu.VMEM((CH,), jnp.int32),
            pltpu.VMEM((CH, D), jnp.float32),
            pltpu.VMEM((CH, D), jnp.float32),
            pltpu.VMEM((ZROWS, D), jnp.float32),
            pltpu.SemaphoreType.DMA,
            pltpu.SemaphoreType.DMA,
            pltpu.VMEM_SHARED((NP, D), jnp.float32),
        ],
        name="sc_rel_agg",
    )


@functools.lru_cache(maxsize=None)
def _make_sc_deg():
    return pl.kernel(
        _sc_deg_body,
        out_type=[jax.ShapeDtypeStruct((R * NP, D), jnp.float32)],
        mesh=plsc.VectorSubcoreMesh(core_axis_name="c", subcore_axis_name="s"),
        scratch_types=[
            pltpu.VMEM((CPT * CH,), jnp.int32),
            pltpu.VMEM((CH,), jnp.int32),
            pltpu.VMEM((CH, D), jnp.float32),
            pltpu.VMEM((ZROWS, D), jnp.float32),
            pltpu.VMEM_SHARED((NP, D), jnp.float32),
        ],
        name="sc_rel_deg",
    )


NBLK = 400


def _tc_mix_body(stage, agg_ref, deg_ref, w_ref, b_ref, o_ref):
    acc = None
    for r in range(R):
        d = jnp.maximum(deg_ref[r, :, 0:1], 1.0)
        a = agg_ref[r] / d
        p = jnp.dot(a, w_ref[r], preferred_element_type=jnp.float32)
        acc = p if acc is None else acc + p
    if stage == 0:
        o_ref[...] = jnp.maximum(acc, 0.0)
    else:
        o_ref[...] = acc + b_ref[...]


def _tc_mix(stage, agg, deg16, W, bias2d):
    return pl.pallas_call(
        functools.partial(_tc_mix_body, stage),
        grid=(N // NBLK,),
        in_specs=[
            pl.BlockSpec((R, NBLK, D), lambda i: (0, i, 0)),
            pl.BlockSpec((R, NBLK, D), lambda i: (0, i, 0)),
            pl.BlockSpec((R, D, D), lambda i: (0, 0, 0)),
            pl.BlockSpec((1, D), lambda i: (0, 0)),
        ],
        out_specs=pl.BlockSpec((NBLK, D), lambda i: (i, 0)),
        out_shape=jax.ShapeDtypeStruct((N, D), jnp.float32),
        name=f"tc_mix{stage}",
    )(agg, deg16, W, bias2d)


def kernel(x, edge_index, W, h_bias):
    src = edge_index[:, 0, :]
    dst = edge_index[:, 1, :]
    srcp = jnp.pad(src, ((0, 0), (0, EPAD - E)))
    dstp = jnp.pad(dst, ((0, 0), (0, EPAD - E)), constant_values=PAD_DST)
    combm = (srcp | (dstp << DST_SHIFT)).reshape(-1)
    bias2d = h_bias.reshape(1, D)

    (deg16,) = _make_sc_deg()(combm)
    deg16 = deg16.reshape(R, NP, D)

    (agg1,) = _make_sc_agg()(x, combm)
    agg1 = agg1.reshape(R, NP, D)
    h = _tc_mix(0, agg1, deg16, W, bias2d)

    (agg2,) = _make_sc_agg()(h, combm)
    agg2 = agg2.reshape(R, NP, D)
    return _tc_mix(1, agg2, deg16, W, bias2d)


# speedup vs baseline: 1.3938x; 1.3938x over previous
import functools

import jax
import jax.numpy as jnp
from jax import lax
from jax.experimental import pallas as pl
from jax.experimental.pallas import tpu as pltpu
from jax.experimental.pallas import tpu_sc as plsc

N = 10000
R = 4
E = 80000
D = 128

NCORE = 2
NSUB = 16
LANES = 16
REL_PER_CORE = R // NCORE
NP = 10240
ROWS_PER_TILE = NP // NSUB
EDGES_PER_TILE = E // NSUB
CH = 128
NCH = EDGES_PER_TILE // CH
TAIL = EDGES_PER_TILE - NCH * CH
ZROWS = 128


def _sc_agg_body(feat, srcf, dstf, agg_out,
                 idx_s, idx_d, rows, z128, tidx_s, tidx_d, trows, sem,
                 acc_sh):
    cid = lax.axis_index("c")
    sid = lax.axis_index("s")

    zv = jnp.zeros((LANES,), jnp.float32)

    def z128_row(i, _):
        def z128_col(j, _):
            z128[i, pl.ds(j * LANES, LANES)] = zv
            return 0
        return lax.fori_loop(0, D // LANES, z128_col, 0)

    lax.fori_loop(0, ZROWS, z128_row, 0)

    row0 = sid * ROWS_PER_TILE

    for k in range(REL_PER_CORE):
        r = cid * REL_PER_CORE + k

        for m in range(ROWS_PER_TILE // ZROWS):
            pltpu.sync_copy(z128, acc_sh.at[pl.ds(row0 + m * ZROWS, ZROWS)])
        plsc.subcore_barrier()

        ebase = r * E + sid * EDGES_PER_TILE

        def chunk(c, _):
            off = ebase + c * CH
            pltpu.sync_copy(srcf.at[pl.ds(off, CH)], idx_s)
            pltpu.sync_copy(dstf.at[pl.ds(off, CH)], idx_d)
            pltpu.async_copy(feat.at[idx_s], rows, sem).wait()
            pltpu.sync_copy(rows, acc_sh.at[idx_d], add=True)
            return 0

        lax.fori_loop(0, NCH, chunk, 0)

        toff = ebase + NCH * CH
        pltpu.sync_copy(srcf.at[pl.ds(toff, TAIL)], tidx_s)
        pltpu.sync_copy(dstf.at[pl.ds(toff, TAIL)], tidx_d)
        pltpu.async_copy(feat.at[tidx_s], trows, sem).wait()
        pltpu.sync_copy(trows, acc_sh.at[tidx_d], add=True)

        plsc.subcore_barrier()

        pltpu.sync_copy(acc_sh.at[pl.ds(row0, ROWS_PER_TILE)],
                        agg_out.at[pl.ds(r * NP + row0, ROWS_PER_TILE)])


def _sc_deg_body(dstf, deg_out, idx_d, ones_b, z128, tidx_d, dacc_sh):
    cid = lax.axis_index("c")
    sid = lax.axis_index("s")

    zv = jnp.zeros((LANES,), jnp.float32)

    def z128_row(i, _):
        def z128_col(j, _):
            z128[i, pl.ds(j * LANES, LANES)] = zv
            return 0
        return lax.fori_loop(0, D // LANES, z128_col, 0)

    lax.fori_loop(0, ZROWS, z128_row, 0)

    ov = jnp.ones((LANES,), jnp.float32)

    def ones_row(i, _):
        def ones_col(j, _):
            ones_b[i, pl.ds(j * LANES, LANES)] = ov
            return 0
        return lax.fori_loop(0, D // LANES, ones_col, 0)

    lax.fori_loop(0, CH, ones_row, 0)

    row0 = sid * ROWS_PER_TILE

    for k in range(REL_PER_CORE):
        r = cid * REL_PER_CORE + k

        for m in range(ROWS_PER_TILE // ZROWS):
            pltpu.sync_copy(z128, dacc_sh.at[pl.ds(row0 + m * ZROWS, ZROWS)])
        plsc.subcore_barrier()

        ebase = r * E + sid * EDGES_PER_TILE

        def chunk(c, _):
            off = ebase + c * CH
            pltpu.sync_copy(dstf.at[pl.ds(off, CH)], idx_d)
            pltpu.sync_copy(ones_b, dacc_sh.at[idx_d], add=True)
            return 0

        lax.fori_loop(0, NCH, chunk, 0)

        toff = ebase + NCH * CH
        pltpu.sync_copy(dstf.at[pl.ds(toff, TAIL)], tidx_d)
        pltpu.sync_copy(ones_b.at[pl.ds(0, TAIL)], dacc_sh.at[tidx_d],
                        add=True)

        plsc.subcore_barrier()

        pltpu.sync_copy(dacc_sh.at[pl.ds(row0, ROWS_PER_TILE)],
                        deg_out.at[pl.ds(r * NP + row0, ROWS_PER_TILE)])


@functools.lru_cache(maxsize=None)
def _make_sc_agg():
    return pl.kernel(
        _sc_agg_body,
        out_type=[jax.ShapeDtypeStruct((R * NP, D), jnp.float32)],
        mesh=plsc.VectorSubcoreMesh(core_axis_name="c", subcore_axis_name="s"),
        scratch_types=[
            pltpu.VMEM((CH,), jnp.int32),
            pltpu.VMEM((CH,), jnp.int32),
            pltpu.VMEM((CH, D), jnp.float32),
            pltpu.VMEM((ZROWS, D), jnp.float32),
            pltpu.VMEM((TAIL,), jnp.int32),
            pltpu.VMEM((TAIL,), jnp.int32),
            pltpu.VMEM((TAIL, D), jnp.float32),
            pltpu.SemaphoreType.DMA,
            pltpu.VMEM_SHARED((NP, D), jnp.float32),
        ],
        name="sc_rel_agg",
    )


@functools.lru_cache(maxsize=None)
def _make_sc_deg():
    return pl.kernel(
        _sc_deg_body,
        out_type=[jax.ShapeDtypeStruct((R * NP, D), jnp.float32)],
        mesh=plsc.VectorSubcoreMesh(core_axis_name="c", subcore_axis_name="s"),
        scratch_types=[
            pltpu.VMEM((CH,), jnp.int32),
            pltpu.VMEM((CH, D), jnp.float32),
            pltpu.VMEM((ZROWS, D), jnp.float32),
            pltpu.VMEM((TAIL,), jnp.int32),
            pltpu.VMEM_SHARED((NP, D), jnp.float32),
        ],
        name="sc_rel_deg",
    )


NBLK = 400


def _tc_mix_body(stage, agg_ref, deg_ref, w_ref, b_ref, o_ref):
    acc = None
    for r in range(R):
        d = jnp.maximum(deg_ref[r, :, 0:1], 1.0)
        a = agg_ref[r] / d
        p = jnp.dot(a, w_ref[r], preferred_element_type=jnp.float32)
        acc = p if acc is None else acc + p
    if stage == 0:
        o_ref[...] = jnp.maximum(acc, 0.0)
    else:
        o_ref[...] = acc + b_ref[...]


def _tc_mix(stage, agg, deg16, W, bias2d):
    return pl.pallas_call(
        functools.partial(_tc_mix_body, stage),
        grid=(N // NBLK,),
        in_specs=[
            pl.BlockSpec((R, NBLK, D), lambda i: (0, i, 0)),
            pl.BlockSpec((R, NBLK, D), lambda i: (0, i, 0)),
            pl.BlockSpec((R, D, D), lambda i: (0, 0, 0)),
            pl.BlockSpec((1, D), lambda i: (0, 0)),
        ],
        out_specs=pl.BlockSpec((NBLK, D), lambda i: (i, 0)),
        out_shape=jax.ShapeDtypeStruct((N, D), jnp.float32),
        name=f"tc_mix{stage}",
    )(agg, deg16, W, bias2d)


def kernel(x, edge_index, W, h_bias):
    src = edge_index[:, 0, :].reshape(-1)
    dst = edge_index[:, 1, :].reshape(-1)
    bias2d = h_bias.reshape(1, D)

    (deg16,) = _make_sc_deg()(dst)
    deg16 = deg16.reshape(R, NP, D)

    (agg1,) = _make_sc_agg()(x, src, dst)
    agg1 = agg1.reshape(R, NP, D)
    h = _tc_mix(0, agg1, deg16, W, bias2d)

    (agg2,) = _make_sc_agg()(h, src, dst)
    agg2 = agg2.reshape(R, NP, D)
    return _tc_mix(1, agg2, deg16, W, bias2d)
